# baseline (speedup 1.0000x reference)
import functools

import jax
import jax.numpy as jnp
from jax import lax
from jax.experimental import pallas as pl
from jax.experimental.pallas import tpu as pltpu
from jax.experimental.pallas import tpu_sc as plsc

N = 10000
FX = 128
B = 64
FU = 64
GO = 128

NW = 32
CW = 312
SUB = 104
NSUB = 3
TAIL = N - NW * CW
NPAD = 10240


def _sc_segment_sums(x, batch_i32):
    mesh = plsc.VectorSubcoreMesh(core_axis_name="c", subcore_axis_name="s")
    ZR = B // 16

    @functools.partial(
        pl.kernel,
        mesh=mesh,
        out_type=jax.ShapeDtypeStruct((2, B, FX), jnp.float32),
        scratch_types=[
            pltpu.VMEM((CW, FX), jnp.float32),
            pltpu.VMEM((NSUB, SUB), jnp.int32),
            pltpu.VMEM((TAIL, FX), jnp.float32),
            pltpu.VMEM((1, TAIL), jnp.int32),
            pltpu.VMEM((ZR, FX), jnp.float32),
            pltpu.VMEM_SHARED((B, FX), jnp.float32),
            pltpu.SemaphoreType.DMA,
            pltpu.SemaphoreType.DMA,
            pltpu.SemaphoreType.DMA,
            pltpu.SemaphoreType.DMA,
            pltpu.SemaphoreType.DMA,
        ],
    )
    def k(x_hbm, b_hbm, sums_out, xv, idxv, xt, idxt, zv, accs,
          sg0, sg1, sg2, si, ss):
        cid = lax.axis_index("c")
        sid = lax.axis_index("s")
        wid = sid * 2 + cid
        base = wid * CW
        sgs = (sg0, sg1, sg2)

        gets = [pltpu.async_copy(x_hbm.at[pl.ds(base + j * SUB, SUB)],
                                 xv.at[pl.ds(j * SUB, SUB)], sgs[j])
                for j in range(NSUB)]
        idx_gets = [pltpu.async_copy(b_hbm.at[pl.ds(base + j * SUB, SUB)],
                                     idxv.at[j], si)
                    for j in range(NSUB)]

        for r in range(ZR):
            for c in range(FX // 16):
                zv[r, pl.ds(c * 16, 16)] = jnp.zeros((16,), jnp.float32)
        pltpu.sync_copy(zv, accs.at[pl.ds(sid * ZR, ZR)])

        for c in idx_gets:
            c.wait()

        plsc.subcore_barrier()

        puts = []
        for j in range(NSUB):
            gets[j].wait()
            puts.append(pltpu.async_copy(xv.at[pl.ds(j * SUB, SUB)],
                                         accs.at[idxv.at[j]], ss, add=True))

        @pl.when(jnp.logical_and(sid == 15, cid == 1))
        def _():
            pltpu.sync_copy(x_hbm.at[pl.ds(NW * CW, TAIL)], xt)
            pltpu.sync_copy(b_hbm.at[pl.ds(NW * CW, TAIL)], idxt.at[0])
            pltpu.sync_copy(xt, accs.at[idxt.at[0]], add=True)

        for c in puts:
            c.wait()

        plsc.subcore_barrier()

        pltpu.sync_copy(accs.at[pl.ds(sid * ZR, ZR)],
                        sums_out.at[cid, pl.ds(sid * ZR, ZR)])

    return k(x, batch_i32)


def _mlp_body(s_ref, b_ref, u_ref,
              g1, bt1, w1, bb1,
              g2, bt2, w2, bb2,
              g3, bt3, w3, bb3,
              o_ref):
    sums = s_ref[0] + s_ref[1]
    t = b_ref[...]
    seg = lax.broadcasted_iota(jnp.int32, (B, N), 0)
    m = jnp.where(t[None, :] == seg, 1.0, 0.0)
    cnt = jnp.sum(m, axis=1, keepdims=True)
    pooled = sums / jnp.maximum(cnt, 1.0)

    def bn(h, g, b):
        mean = jnp.mean(h, axis=0, keepdims=True)
        v = jnp.mean((h - mean) ** 2, axis=0, keepdims=True)
        return (h - mean) * lax.rsqrt(v + 1e-5) * g.reshape(1, -1) \
            + b.reshape(1, -1)

    def dot_t(h, w):
        return lax.dot_general(h, w, (((1,), (1,)), ((), ())),
                               preferred_element_type=jnp.float32)

    def elu(h):
        return jnp.where(h > 0, h, jnp.exp(jnp.minimum(h, 0.0)) - 1.0)

    h = jnp.concatenate([u_ref[...], pooled], axis=1)
    h = bn(h, g1[...], bt1[...])
    h = lax.dot_general(h, w1[...], (((1,), (0,)), ((), ())),
                        preferred_element_type=jnp.float32) \
        + bb1[...].reshape(1, -1)
    h = elu(h)
    h = bn(h, g2[...], bt2[...])
    h = dot_t(h, w2[...]) + bb2[...].reshape(1, -1)
    h = elu(h)
    h = bn(h, g3[...], bt3[...])
    o_ref[...] = dot_t(h, w3[...]) + bb3[...].reshape(1, -1)


def kernel(x, edge_index, edge_attr, u, batch,
           bn1_g, bn1_b, W1, b1,
           bn2_g, bn2_b, W2, b2,
           bn3_g, bn3_b, W3, b3):
    del edge_index, edge_attr
    batch_i32 = batch if batch.dtype == jnp.int32 else batch.astype(jnp.int32)

    sums2 = _sc_segment_sums(x, batch_i32)

    return pl.pallas_call(
        _mlp_body,
        out_shape=jax.ShapeDtypeStruct((B, GO), jnp.float32),
    )(sums2, batch_i32, u,
      bn1_g, bn1_b, W1.T, b1,
      bn2_g, bn2_b, W2, b2,
      bn3_g, bn3_b, W3, b3)


# speedup vs baseline: 1.0159x; 1.0159x over previous
import functools

import jax
import jax.numpy as jnp
from jax import lax
from jax.experimental import pallas as pl
from jax.experimental.pallas import tpu as pltpu
from jax.experimental.pallas import tpu_sc as plsc

N = 10000
FX = 128
B = 64
FU = 64
GO = 128

NW = 32
CW = 312
SUB = 104
NSUB = 3
TAIL = N - NW * CW
NPAD = 10240


def _sc_segment_sums(x, batch_i32):
    mesh = plsc.VectorSubcoreMesh(core_axis_name="c", subcore_axis_name="s")
    ZR = B // 16

    @functools.partial(
        pl.kernel,
        mesh=mesh,
        out_type=jax.ShapeDtypeStruct((2, B, FX), jnp.float32),
        scratch_types=[
            pltpu.VMEM((CW, FX), jnp.float32),
            pltpu.VMEM((NSUB, SUB), jnp.int32),
            pltpu.VMEM((TAIL, FX), jnp.float32),
            pltpu.VMEM((1, TAIL), jnp.int32),
            pltpu.VMEM((ZR, FX), jnp.float32),
            pltpu.VMEM_SHARED((B, FX), jnp.float32),
            pltpu.SemaphoreType.DMA,
            pltpu.SemaphoreType.DMA,
            pltpu.SemaphoreType.DMA,
            pltpu.SemaphoreType.DMA,
            pltpu.SemaphoreType.DMA,
        ],
    )
    def k(x_hbm, b_hbm, sums_out, xv, idxv, xt, idxt, zv, accs,
          sg0, sg1, sg2, si, ss):
        cid = lax.axis_index("c")
        sid = lax.axis_index("s")
        wid = sid * 2 + cid
        base = wid * CW
        sgs = (sg0, sg1, sg2)

        gets = [pltpu.async_copy(x_hbm.at[pl.ds(base + j * SUB, SUB)],
                                 xv.at[pl.ds(j * SUB, SUB)], sgs[j])
                for j in range(NSUB)]
        idx_gets = [pltpu.async_copy(b_hbm.at[pl.ds(base + j * SUB, SUB)],
                                     idxv.at[j], si)
                    for j in range(NSUB)]

        for r in range(ZR):
            for c in range(FX // 16):
                zv[r, pl.ds(c * 16, 16)] = jnp.zeros((16,), jnp.float32)
        pltpu.sync_copy(zv, accs.at[pl.ds(sid * ZR, ZR)])

        for c in idx_gets:
            c.wait()

        plsc.subcore_barrier()

        puts = []
        for j in range(NSUB):
            gets[j].wait()
            puts.append(pltpu.async_copy(xv.at[pl.ds(j * SUB, SUB)],
                                         accs.at[idxv.at[j]], ss, add=True))

        @pl.when(jnp.logical_and(sid == 15, cid == 1))
        def _():
            pltpu.sync_copy(x_hbm.at[pl.ds(NW * CW, TAIL)], xt)
            pltpu.sync_copy(b_hbm.at[pl.ds(NW * CW, TAIL)], idxt.at[0])
            pltpu.sync_copy(xt, accs.at[idxt.at[0]], add=True)

        for c in puts:
            c.wait()

        plsc.subcore_barrier()

        pltpu.sync_copy(accs.at[pl.ds(sid * ZR, ZR)],
                        sums_out.at[cid, pl.ds(sid * ZR, ZR)])

    return k(x, batch_i32)


def _counts_body(b_ref, o_ref):
    t = b_ref[...]
    seg = lax.broadcasted_iota(jnp.int32, (B, N), 0)
    m = jnp.where(t[None, :] == seg, 1.0, 0.0)
    o_ref[...] = jnp.sum(m, axis=1, keepdims=True)


def _mlp_body(s_ref, c_ref, u_ref,
              g1, bt1, w1, bb1,
              g2, bt2, w2, bb2,
              g3, bt3, w3, bb3,
              o_ref):
    sums = s_ref[0] + s_ref[1]
    cnt = c_ref[...]
    pooled = sums / jnp.maximum(cnt, 1.0)

    def bn(h, g, b):
        mean = jnp.mean(h, axis=0, keepdims=True)
        v = jnp.mean((h - mean) ** 2, axis=0, keepdims=True)
        return (h - mean) * lax.rsqrt(v + 1e-5) * g.reshape(1, -1) \
            + b.reshape(1, -1)

    def dot_t(h, w):
        return lax.dot_general(h, w, (((1,), (1,)), ((), ())),
                               preferred_element_type=jnp.float32)

    def elu(h):
        return jnp.where(h > 0, h, jnp.exp(jnp.minimum(h, 0.0)) - 1.0)

    h = jnp.concatenate([u_ref[...], pooled], axis=1)
    h = bn(h, g1[...], bt1[...])
    h = lax.dot_general(h, w1[...], (((1,), (0,)), ((), ())),
                        preferred_element_type=jnp.float32) \
        + bb1[...].reshape(1, -1)
    h = elu(h)
    h = bn(h, g2[...], bt2[...])
    h = dot_t(h, w2[...]) + bb2[...].reshape(1, -1)
    h = elu(h)
    h = bn(h, g3[...], bt3[...])
    o_ref[...] = dot_t(h, w3[...]) + bb3[...].reshape(1, -1)


def kernel(x, edge_index, edge_attr, u, batch,
           bn1_g, bn1_b, W1, b1,
           bn2_g, bn2_b, W2, b2,
           bn3_g, bn3_b, W3, b3):
    del edge_index, edge_attr
    batch_i32 = batch if batch.dtype == jnp.int32 else batch.astype(jnp.int32)

    cnt = pl.pallas_call(
        _counts_body,
        out_shape=jax.ShapeDtypeStruct((B, 1), jnp.float32),
    )(batch_i32)

    sums2 = _sc_segment_sums(x, batch_i32)

    return pl.pallas_call(
        _mlp_body,
        out_shape=jax.ShapeDtypeStruct((B, GO), jnp.float32),
    )(sums2, cnt, u,
      bn1_g, bn1_b, W1.T, b1,
      bn2_g, bn2_b, W2, b2,
      bn3_g, bn3_b, W3, b3)
